# baseline (speedup 1.0000x reference)
import functools

import jax
import jax.numpy as jnp
from jax import lax
from jax.experimental import pallas as pl
from jax.experimental.pallas import tpu as pltpu
from jax.experimental.pallas import tpu_sc as plsc

NC = 2
NS = 16
NW = NC * NS

B, L, W = 4096, 200, 32
TOTAL = B * L
SUB = 128
NSUB = 10
CHUNK = SUB * NSUB
NCHUNKS = TOTAL // CHUNK
CPW = NCHUNKS // NW
PAD_ROW = 2048


def _sc_body(ins_hbm, table_hbm, out_hbm,
             in0, in1, idxm0, idxm1, rows0, rows1,
             insem0, insem1, gsem, outsem0, outsem1):
    wid = lax.axis_index("s") * NC + lax.axis_index("c")
    base = wid * CPW
    in_bufs = (in0, in1)
    idxm_bufs = (idxm0, idxm1)
    row_bufs = (rows0, rows1)
    insems = (insem0, insem1)
    outsems = (outsem0, outsem1)

    def start_in(cid, slot):
        pltpu.async_copy(ins_hbm.at[cid], in_bufs[slot], insems[slot])

    start_in(base + 0, 0)
    start_in(base + 1, 1)

    def do_chunk(t, c, slot):
        in_v = in_bufs[slot]
        idxm_v = idxm_bufs[slot]
        rows_v = row_bufs[slot]
        pltpu.make_async_copy(ins_hbm.at[0], in_v, insems[slot]).wait()
        for j in range(NSUB):
            for i in range(SUB // 16):
                sl = pl.ds(i * 16, 16)
                m = in_v[1, j, sl]
                x = in_v[0, j, sl]
                idxm_v[j, sl] = jnp.where(m != 0, x, PAD_ROW)

        @pl.when(c + 2 < base + CPW)
        def _():
            start_in(c + 2, slot)

        @pl.when(t > 0)
        def _():
            pltpu.make_async_copy(rows_v, out_hbm.at[c], outsems[slot]).wait()

        cps = [
            pltpu.async_copy(table_hbm.at[idxm_v.at[j]], rows_v.at[j], gsem)
            for j in range(NSUB)
        ]
        for cp in cps:
            cp.wait()
        pltpu.async_copy(rows_v, out_hbm.at[c], outsems[slot])

    def body(t, carry):
        do_chunk(t, base + 2 * t, 0)
        do_chunk(t, base + 2 * t + 1, 1)
        return carry

    lax.fori_loop(0, CPW // 2, body, 0)
    pltpu.make_async_copy(rows0, out_hbm.at[base], outsems[0]).wait()
    pltpu.make_async_copy(rows1, out_hbm.at[base], outsems[1]).wait()


@jax.jit
def _run(ins3, table_pad):
    mesh = plsc.VectorSubcoreMesh(core_axis_name="c", subcore_axis_name="s")
    kfn = pl.kernel(
        _sc_body,
        out_type=jax.ShapeDtypeStruct((NCHUNKS, NSUB, SUB, W), jnp.float32),
        mesh=mesh,
        scratch_types=[
            pltpu.VMEM((2, NSUB, SUB), jnp.int32),
            pltpu.VMEM((2, NSUB, SUB), jnp.int32),
            pltpu.VMEM((NSUB, SUB), jnp.int32),
            pltpu.VMEM((NSUB, SUB), jnp.int32),
            pltpu.VMEM((NSUB, SUB, W), jnp.float32),
            pltpu.VMEM((NSUB, SUB, W), jnp.float32),
            pltpu.SemaphoreType.DMA,
            pltpu.SemaphoreType.DMA,
            pltpu.SemaphoreType.DMA,
            pltpu.SemaphoreType.DMA,
            pltpu.SemaphoreType.DMA,
        ],
        compiler_params=pltpu.CompilerParams(use_tc_tiling_on_sc=False),
    )
    return kfn(ins3, table_pad)


def kernel(obs_pos, obs_mask, embedding_table):
    idx3 = obs_pos.astype(jnp.int32).reshape(NCHUNKS, NSUB, SUB)
    mask3 = obs_mask.astype(jnp.int32).reshape(NCHUNKS, NSUB, SUB)
    ins3 = jnp.stack([idx3, mask3], axis=1)
    table_pad = jnp.concatenate(
        [embedding_table, jnp.zeros((8, W), jnp.float32)], axis=0)
    out = _run(ins3, table_pad)
    return out.reshape(B, L, W)


# speedup vs baseline: 7.4856x; 7.4856x over previous
import jax
import jax.numpy as jnp
from jax import lax
from jax.experimental import pallas as pl
from jax.experimental.pallas import tpu as pltpu
from jax.experimental.pallas import tpu_sc as plsc

NC = 2
NS = 16
NW = NC * NS

B, L, W = 4096, 200, 32
TOTAL = B * L
SUB = 128
NSUB = 5
CHUNK = SUB * NSUB
NCHUNKS = TOTAL // CHUNK
CPW = NCHUNKS // NW
TROWS = 2056
PAD_ROW = 2048


def _sc_body(ins_hbm, table_hbm, out_hbm,
             table_v, in0, in1, idxm0, idxm1, rows0, rows1,
             insem0, insem1, gsem, outsem0, outsem1):
    wid = lax.axis_index("s") * NC + lax.axis_index("c")
    base = wid * CPW
    in_bufs = (in0, in1)
    idxm_bufs = (idxm0, idxm1)
    row_bufs = (rows0, rows1)
    insems = (insem0, insem1)
    outsems = (outsem0, outsem1)

    def start_in(cid, slot):
        pltpu.async_copy(ins_hbm.at[cid], in_bufs[slot], insems[slot])

    start_in(base + 0, 0)
    start_in(base + 1, 1)

    @pl.when(lax.axis_index("s") == 0)
    def _():
        pltpu.sync_copy(table_hbm, table_v)

    plsc.subcore_barrier()

    def do_chunk(t, c, slot):
        in_v = in_bufs[slot]
        idxm_v = idxm_bufs[slot]
        rows_v = row_bufs[slot]
        pltpu.make_async_copy(ins_hbm.at[0], in_v, insems[slot]).wait()
        for j in range(NSUB):
            for i in range(SUB // 16):
                sl = pl.ds(i * 16, 16)
                m = in_v[1, j, sl]
                x = in_v[0, j, sl]
                idxm_v[j, sl] = jnp.where(m != 0, x, PAD_ROW)

        @pl.when(c + 2 < base + CPW)
        def _():
            start_in(c + 2, slot)

        @pl.when(t > 0)
        def _():
            pltpu.make_async_copy(rows_v, out_hbm.at[c], outsems[slot]).wait()

        cps = [
            pltpu.async_copy(table_v.at[idxm_v.at[j]], rows_v.at[j], gsem)
            for j in range(NSUB)
        ]
        for cp in cps:
            cp.wait()
        pltpu.async_copy(rows_v, out_hbm.at[c], outsems[slot])

    def body(t, carry):
        do_chunk(t, base + 2 * t, 0)
        do_chunk(t, base + 2 * t + 1, 1)
        return carry

    lax.fori_loop(0, CPW // 2, body, 0)
    pltpu.make_async_copy(rows0, out_hbm.at[base], outsems[0]).wait()
    pltpu.make_async_copy(rows1, out_hbm.at[base], outsems[1]).wait()


@jax.jit
def _run(ins3, table_pad):
    mesh = plsc.VectorSubcoreMesh(core_axis_name="c", subcore_axis_name="s")
    kfn = pl.kernel(
        _sc_body,
        out_type=jax.ShapeDtypeStruct((NCHUNKS, NSUB, SUB, W), jnp.float32),
        mesh=mesh,
        scratch_types=[
            pltpu.VMEM_SHARED((TROWS, W), jnp.float32),
            pltpu.VMEM((2, NSUB, SUB), jnp.int32),
            pltpu.VMEM((2, NSUB, SUB), jnp.int32),
            pltpu.VMEM((NSUB, SUB), jnp.int32),
            pltpu.VMEM((NSUB, SUB), jnp.int32),
            pltpu.VMEM((NSUB, SUB, W), jnp.float32),
            pltpu.VMEM((NSUB, SUB, W), jnp.float32),
            pltpu.SemaphoreType.DMA,
            pltpu.SemaphoreType.DMA,
            pltpu.SemaphoreType.DMA,
            pltpu.SemaphoreType.DMA,
            pltpu.SemaphoreType.DMA,
        ],
        compiler_params=pltpu.CompilerParams(use_tc_tiling_on_sc=False),
    )
    return kfn(ins3, table_pad)


def kernel(obs_pos, obs_mask, embedding_table):
    idx3 = obs_pos.astype(jnp.int32).reshape(NCHUNKS, NSUB, SUB)
    mask3 = obs_mask.astype(jnp.int32).reshape(NCHUNKS, NSUB, SUB)
    ins3 = jnp.stack([idx3, mask3], axis=1)
    table_pad = jnp.concatenate(
        [embedding_table, jnp.zeros((TROWS - 2048, W), jnp.float32)], axis=0)
    out = _run(ins3, table_pad)
    return out.reshape(B, L, W)
